# Initial kernel scaffold; baseline (speedup 1.0000x reference)
import jax
import jax.numpy as jnp
from jax.experimental import pallas as pl


def kernel(input_ids, word_embeddings, text_embeddings):
    raise NotImplementedError("write your pallas kernel here")


# speedup vs baseline: 2.9408x; 2.9408x over previous
import functools

import jax
import jax.numpy as jnp
from jax import lax
from jax.experimental import pallas as pl
from jax.experimental.pallas import tpu as pltpu
from jax.experimental.pallas import tpu_sc as plsc

D = 128
B_TOK = 1024 * 512
NC, NS = 2, 16
NW = NC * NS
K = 128
CHUNKS = B_TOK // (NW * K)


def _body(idx_hbm, table_hbm, out_hbm, idx_v, rows_v, gsem):
    wid = lax.axis_index("s") * NC + lax.axis_index("c")
    pltpu.sync_copy(idx_hbm.at[wid], idx_v)
    base = wid * (CHUNKS * K)

    def step(j, carry):
        pltpu.async_copy(table_hbm.at[idx_v.at[j]], rows_v, gsem).wait()
        pltpu.sync_copy(rows_v, out_hbm.at[pl.ds(base + j * K, K)])
        return carry

    lax.fori_loop(0, CHUNKS, step, 0)


def kernel(input_ids, word_embeddings, text_embeddings):
    del text_embeddings
    idx = input_ids.reshape(NW, CHUNKS, K)
    run = functools.partial(
        pl.kernel,
        mesh=plsc.VectorSubcoreMesh(core_axis_name="c", subcore_axis_name="s"),
        out_type=jax.ShapeDtypeStruct((B_TOK, D), jnp.float32),
        scratch_types=[
            pltpu.VMEM((CHUNKS, K), jnp.int32),
            pltpu.VMEM((K, D), jnp.float32),
            pltpu.SemaphoreType.DMA,
        ],
    )(_body)
    out = run(idx, word_embeddings)
    return out.reshape(1024, 512, D)


# speedup vs baseline: 2.9533x; 1.0043x over previous
import functools

import jax
import jax.numpy as jnp
from jax import lax
from jax.experimental import pallas as pl
from jax.experimental.pallas import tpu as pltpu
from jax.experimental.pallas import tpu_sc as plsc

D = 128
B_TOK = 1024 * 512
NC, NS = 2, 16
NW = NC * NS
K = 128
CHUNKS = B_TOK // (NW * K)
NBUF = 4


def _body(idx_hbm, table_hbm, out_hbm, idx_v, rows_v, gsem, wsem):
    wid = lax.axis_index("s") * NC + lax.axis_index("c")
    pltpu.sync_copy(idx_hbm.at[wid], idx_v)
    base = wid * (CHUNKS * K)

    def gather_copy(j, b):
        return pltpu.make_async_copy(table_hbm.at[idx_v.at[j]],
                                     rows_v.at[b], gsem)

    def write_copy(j, b):
        return pltpu.make_async_copy(rows_v.at[b],
                                     out_hbm.at[pl.ds(base + j * K, K)],
                                     wsem)

    gather_copy(0, 0).start()
    gather_copy(1, 1).start()
    gather_copy(2, 2).start()
    gather_copy(0, 0).wait()
    write_copy(0, 0).start()
    gather_copy(3, 3).start()
    gather_copy(1, 1).wait()
    write_copy(1, 1).start()

    def outer(i, carry):
        for b in range(NBUF):
            j = i * NBUF + b
            write_copy(j - NBUF, b).wait()
            gather_copy(j, b).start()
            pb = (b + 2) % NBUF
            gather_copy(j - 2, pb).wait()
            write_copy(j - 2, pb).start()
        return carry

    lax.fori_loop(1, CHUNKS // NBUF, outer, 0)

    gather_copy(CHUNKS - 2, 2).wait()
    write_copy(CHUNKS - 2, 2).start()
    gather_copy(CHUNKS - 1, 3).wait()
    write_copy(CHUNKS - 1, 3).start()
    for b in range(NBUF):
        write_copy(CHUNKS - NBUF + b, b).wait()


def kernel(input_ids, word_embeddings, text_embeddings):
    del text_embeddings
    idx = input_ids.reshape(NW, CHUNKS, K)
    run = functools.partial(
        pl.kernel,
        mesh=plsc.VectorSubcoreMesh(core_axis_name="c", subcore_axis_name="s"),
        out_type=jax.ShapeDtypeStruct((B_TOK, D), jnp.float32),
        scratch_types=[
            pltpu.VMEM((CHUNKS, K), jnp.int32),
            pltpu.VMEM((NBUF, K, D), jnp.float32),
            pltpu.SemaphoreType.DMA,
            pltpu.SemaphoreType.DMA,
        ],
    )(_body)
    out = run(idx, word_embeddings)
    return out.reshape(1024, 512, D)


# speedup vs baseline: 15.4749x; 5.2398x over previous
import functools

import jax
import jax.numpy as jnp
from jax import lax
from jax.experimental import pallas as pl
from jax.experimental.pallas import tpu as pltpu
from jax.experimental.pallas import tpu_sc as plsc

D = 128
B_TOK = 1024 * 512
NC, NS = 2, 16
NW = NC * NS
K = 128
CHUNKS = B_TOK // (NW * K)
NBUF = 4


def _body(idx_hbm, table_hbm, out_hbm, idx_v, rows_v, tab_v, tab_sh,
          gsem, wsem):
    sid = lax.axis_index("s")
    wid = sid * NC + lax.axis_index("c")

    @pl.when(sid == 0)
    def _():
        pltpu.sync_copy(table_hbm, tab_v)
        pltpu.sync_copy(tab_v, tab_sh)

    pltpu.sync_copy(idx_hbm.at[wid], idx_v)
    plsc.subcore_barrier()
    base = wid * (CHUNKS * K)

    def gather_copy(j, b):
        return pltpu.make_async_copy(tab_sh.at[idx_v.at[j]],
                                     rows_v.at[b], gsem)

    def write_copy(j, b):
        return pltpu.make_async_copy(rows_v.at[b],
                                     out_hbm.at[pl.ds(base + j * K, K)],
                                     wsem)

    gather_copy(0, 0).start()
    gather_copy(1, 1).start()
    gather_copy(2, 2).start()
    gather_copy(0, 0).wait()
    write_copy(0, 0).start()
    gather_copy(3, 3).start()
    gather_copy(1, 1).wait()
    write_copy(1, 1).start()

    def outer(i, carry):
        for b in range(NBUF):
            j = i * NBUF + b
            write_copy(j - NBUF, b).wait()
            gather_copy(j, b).start()
            pb = (b + 2) % NBUF
            gather_copy(j - 2, pb).wait()
            write_copy(j - 2, pb).start()
        return carry

    lax.fori_loop(1, CHUNKS // NBUF, outer, 0)

    gather_copy(CHUNKS - 2, 2).wait()
    write_copy(CHUNKS - 2, 2).start()
    gather_copy(CHUNKS - 1, 3).wait()
    write_copy(CHUNKS - 1, 3).start()
    for b in range(NBUF):
        write_copy(CHUNKS - NBUF + b, b).wait()


def kernel(input_ids, word_embeddings, text_embeddings):
    del text_embeddings
    idx = input_ids.reshape(NW, CHUNKS, K)
    run = functools.partial(
        pl.kernel,
        mesh=plsc.VectorSubcoreMesh(core_axis_name="c", subcore_axis_name="s"),
        out_type=jax.ShapeDtypeStruct((B_TOK, D), jnp.float32),
        scratch_types=[
            pltpu.VMEM((CHUNKS, K), jnp.int32),
            pltpu.VMEM((NBUF, K, D), jnp.float32),
            pltpu.VMEM((178, D), jnp.float32),
            pltpu.VMEM_SHARED((178, D), jnp.float32),
            pltpu.SemaphoreType.DMA,
            pltpu.SemaphoreType.DMA,
        ],
    )(_body)
    out = run(idx, word_embeddings)
    return out.reshape(1024, 512, D)
